# baseline (speedup 1.0000x reference)
import jax
import jax.numpy as jnp
from jax import lax
from jax.experimental import pallas as pl
from jax.experimental.pallas import tpu as pltpu
from jax.experimental.pallas import tpu_sc as plsc

N = 10000
D = 128
NC, NS = 2, 16
NW = NC * NS
CHUNK = 128
GCHUNK = 256
SCHUNK = 128
NP = 16384
ACC_N = 10112

_mesh = plsc.VectorSubcoreMesh(core_axis_name="c", subcore_axis_name="s")


def _wid():
    return lax.axis_index("s") * NC + lax.axis_index("c")


def _emb_body(n0, n1, e0, e1, table, nf_out, ef_out,
              i0a, i1a, i0b, i1b, r0a, r1a, r0b, r1b, semA, semB, semW):
    w = _wid()
    n_edge_chunks = ef_out.shape[0] // (NW * CHUNK)

    def mean_rows(ra, rb):
        def row(r, _):
            for cc in range(8):
                sl = pl.ds(cc * 16, 16)
                ra[r, sl] = (ra[r, sl] + rb[r, sl]) * 0.5
            return 0

        lax.fori_loop(0, CHUNK, row, 0)

    def do_part(idx0_hbm, idx1_hbm, out_hbm, n_chunks):
        def pair(p, _):
            ia = w * n_chunks + 2 * p
            ib = ia + 1
            pltpu.sync_copy(idx0_hbm.at[pl.ds(ia, 1)], i0a)
            pltpu.sync_copy(idx1_hbm.at[pl.ds(ia, 1)], i1a)
            pltpu.sync_copy(idx0_hbm.at[pl.ds(ib, 1)], i0b)
            pltpu.sync_copy(idx1_hbm.at[pl.ds(ib, 1)], i1b)
            hA0 = pltpu.async_copy(table.at[i0a.at[0]], r0a, semA)
            hA1 = pltpu.async_copy(table.at[i1a.at[0]], r1a, semA)
            hB0 = pltpu.async_copy(table.at[i0b.at[0]], r0b, semB)
            hB1 = pltpu.async_copy(table.at[i1b.at[0]], r1b, semB)
            hA0.wait()
            hA1.wait()
            mean_rows(r0a, r1a)
            wA = pltpu.async_copy(r0a, out_hbm.at[pl.ds(ia * CHUNK, CHUNK)], semW)
            hB0.wait()
            hB1.wait()
            mean_rows(r0b, r1b)
            wB = pltpu.async_copy(r0b, out_hbm.at[pl.ds(ib * CHUNK, CHUNK)], semW)
            wA.wait()
            wB.wait()
            return 0

        lax.fori_loop(0, n_chunks // 2, pair, 0)

    do_part(n0, n1, nf_out, NP // (NW * CHUNK))
    do_part(e0, e1, ef_out, n_edge_chunks)


def _gather_body(src2d, table, g_out, iva, ivb, rva, rvb, semA, semB, semW):
    w = _wid()
    n_chunks = g_out.shape[0] // (NW * GCHUNK)
    k = GCHUNK // 128

    def pair(p, _):
        ia = w * n_chunks + 2 * p
        ib = ia + 1
        pltpu.sync_copy(src2d.at[pl.ds(ia * k, k)], iva)
        pltpu.sync_copy(src2d.at[pl.ds(ib * k, k)], ivb)
        hA = [pltpu.async_copy(table.at[iva.at[j]], rva.at[pl.ds(j * 128, 128)], semA)
              for j in range(k)]
        hB = [pltpu.async_copy(table.at[ivb.at[j]], rvb.at[pl.ds(j * 128, 128)], semB)
              for j in range(k)]
        for h in hA:
            h.wait()
        wA = pltpu.async_copy(rva, g_out.at[pl.ds(ia * GCHUNK, GCHUNK)], semW)
        for h in hB:
            h.wait()
        wB = pltpu.async_copy(rvb, g_out.at[pl.ds(ib * GCHUNK, GCHUNK)], semW)
        wA.wait()
        wB.wait()
        return 0

    lax.fori_loop(0, n_chunks // 2, pair, 0)


def _scatter_body(dst2d, msgs, agg_out, iva, ivb, rva, rvb, semA, semB, acc):
    c = lax.axis_index("c")
    s = lax.axis_index("s")
    w = _wid()
    n_chunks = msgs.shape[0] // (NW * SCHUNK)
    per_tile = ACC_N // NS

    def zrow(r, _):
        for cc in range(8):
            rva[r, pl.ds(cc * 16, 16)] = jnp.zeros((16,), jnp.float32)
        return 0

    lax.fori_loop(0, 128, zrow, 0)
    for t in range(per_tile // 128):
        pltpu.sync_copy(rva.at[pl.ds(0, 128)],
                        acc.at[pl.ds(s * per_tile + t * 128, 128)])
    rem = per_tile % 128
    if rem:
        pltpu.sync_copy(rva.at[pl.ds(0, rem)],
                        acc.at[pl.ds(s * per_tile + (per_tile // 128) * 128, rem)])
    plsc.subcore_barrier()

    def pair(p, _):
        ia = w * n_chunks + 2 * p
        ib = ia + 1
        pltpu.sync_copy(dst2d.at[pl.ds(ia, 1)], iva)
        pltpu.sync_copy(dst2d.at[pl.ds(ib, 1)], ivb)
        lA = pltpu.async_copy(msgs.at[pl.ds(ia * SCHUNK, SCHUNK)], rva, semA)
        lB = pltpu.async_copy(msgs.at[pl.ds(ib * SCHUNK, SCHUNK)], rvb, semB)
        lA.wait()
        sA = pltpu.async_copy(rva, acc.at[iva.at[0]], semA, add=True)
        lB.wait()
        sB = pltpu.async_copy(rvb, acc.at[ivb.at[0]], semB, add=True)
        sA.wait()
        sB.wait()
        return 0

    lax.fori_loop(0, n_chunks // 2, pair, 0)
    plsc.subcore_barrier()
    pltpu.sync_copy(acc.at[pl.ds(s * per_tile, per_tile)],
                    agg_out.at[c, pl.ds(s * per_tile, per_tile)])


def _edge_tc_body(g_ref, ef_ref, wm_ref, we_ref, msgs_ref, efn_ref):
    ef = ef_ref[...]
    m = jnp.maximum(
        jnp.dot(g_ref[...] + ef, wm_ref[...], preferred_element_type=jnp.float32), 0.0)
    msgs_ref[...] = m
    efn_ref[...] = jnp.maximum(
        jnp.dot(ef, we_ref[...], preferred_element_type=jnp.float32) + m, 0.0)


def _node_tc_body(nf_ref, ws_ref, agg_ref, out_ref):
    acc = agg_ref[0] + agg_ref[1]
    out_ref[...] = jnp.maximum(
        jnp.dot(nf_ref[...], ws_ref[...], preferred_element_type=jnp.float32) + acc, 0.0)


def _node_final_body(_aliased_ref, nf_ref, ws_ref, agg_ref, out_ref):
    acc = agg_ref[0] + agg_ref[1]
    out_ref[...] = jnp.maximum(
        jnp.dot(nf_ref[...], ws_ref[...], preferred_element_type=jnp.float32) + acc, 0.0)


BE = 2000


def _edge_tc(g, ef, Wm, We, E):
    EP = g.shape[0]
    return pl.pallas_call(
        _edge_tc_body,
        grid=(E // BE,),
        in_specs=[
            pl.BlockSpec((BE, D), lambda i: (i, 0)),
            pl.BlockSpec((BE, D), lambda i: (i, 0)),
            pl.BlockSpec((D, D), lambda i: (0, 0)),
            pl.BlockSpec((D, D), lambda i: (0, 0)),
        ],
        out_specs=[
            pl.BlockSpec((BE, D), lambda i: (i, 0)),
            pl.BlockSpec((BE, D), lambda i: (i, 0)),
        ],
        out_shape=[
            jax.ShapeDtypeStruct((EP, D), jnp.float32),
            jax.ShapeDtypeStruct((E, D), jnp.float32),
        ],
        compiler_params=pltpu.CompilerParams(dimension_semantics=("parallel",)),
    )(g, ef, Wm, We)


def _edge_tc_final(g, ef, Wm, We, E, T):
    EP = g.shape[0]
    nblk = N // BE
    return pl.pallas_call(
        _edge_tc_body,
        grid=(E // BE,),
        in_specs=[
            pl.BlockSpec((BE, D), lambda i: (i, 0)),
            pl.BlockSpec((BE, D), lambda i: (i, 0)),
            pl.BlockSpec((D, D), lambda i: (0, 0)),
            pl.BlockSpec((D, D), lambda i: (0, 0)),
        ],
        out_specs=[
            pl.BlockSpec((BE, D), lambda i: (i, 0)),
            pl.BlockSpec((BE, D), lambda i: (i + nblk, 0)),
        ],
        out_shape=[
            jax.ShapeDtypeStruct((EP, D), jnp.float32),
            jax.ShapeDtypeStruct((T, D), jnp.float32),
        ],
        compiler_params=pltpu.CompilerParams(dimension_semantics=("parallel",)),
    )(g, ef, Wm, We)


def _node_tc(nf_pad, Ws, agg):
    BN = 2000
    return pl.pallas_call(
        _node_tc_body,
        grid=(N // BN,),
        in_specs=[
            pl.BlockSpec((BN, D), lambda i: (i, 0)),
            pl.BlockSpec((D, D), lambda i: (0, 0)),
            pl.BlockSpec((2, BN, D), lambda i: (0, i, 0)),
        ],
        out_specs=pl.BlockSpec((BN, D), lambda i: (i, 0)),
        out_shape=jax.ShapeDtypeStruct((N, D), jnp.float32),
        compiler_params=pltpu.CompilerParams(dimension_semantics=("parallel",)),
    )(nf_pad, Ws, agg)


def _node_tc_final(allfeats, nf, Ws, agg):
    BN = 2000
    T = allfeats.shape[0]
    return pl.pallas_call(
        _node_final_body,
        grid=(N // BN,),
        in_specs=[
            pl.BlockSpec((BN, D), lambda i: (i, 0)),
            pl.BlockSpec((BN, D), lambda i: (i, 0)),
            pl.BlockSpec((D, D), lambda i: (0, 0)),
            pl.BlockSpec((2, BN, D), lambda i: (0, i, 0)),
        ],
        out_specs=pl.BlockSpec((BN, D), lambda i: (i, 0)),
        out_shape=jax.ShapeDtypeStruct((T, D), jnp.float32),
        input_output_aliases={0: 0},
        compiler_params=pltpu.CompilerParams(dimension_semantics=("parallel",)),
    )(allfeats, nf, Ws, agg)


def _sc_emb(n0, n1, e0, e1, W_embed, EP):
    call = pl.kernel(
        _emb_body,
        out_type=[
            jax.ShapeDtypeStruct((NP, D), jnp.float32),
            jax.ShapeDtypeStruct((EP, D), jnp.float32),
        ],
        mesh=_mesh,
        scratch_types=[
            pltpu.VMEM((1, 128), jnp.int32),
            pltpu.VMEM((1, 128), jnp.int32),
            pltpu.VMEM((1, 128), jnp.int32),
            pltpu.VMEM((1, 128), jnp.int32),
            pltpu.VMEM((CHUNK, D), jnp.float32),
            pltpu.VMEM((CHUNK, D), jnp.float32),
            pltpu.VMEM((CHUNK, D), jnp.float32),
            pltpu.VMEM((CHUNK, D), jnp.float32),
            pltpu.SemaphoreType.DMA,
            pltpu.SemaphoreType.DMA,
            pltpu.SemaphoreType.DMA,
        ],
    )
    return call(n0, n1, e0, e1, W_embed)


def _sc_gather(src2d, table, EP):
    call = pl.kernel(
        _gather_body,
        out_type=jax.ShapeDtypeStruct((EP, D), jnp.float32),
        mesh=_mesh,
        scratch_types=[
            pltpu.VMEM((GCHUNK // 128, 128), jnp.int32),
            pltpu.VMEM((GCHUNK // 128, 128), jnp.int32),
            pltpu.VMEM((GCHUNK, D), jnp.float32),
            pltpu.VMEM((GCHUNK, D), jnp.float32),
            pltpu.SemaphoreType.DMA,
            pltpu.SemaphoreType.DMA,
            pltpu.SemaphoreType.DMA,
        ],
    )
    return call(src2d, table)


def _sc_scatter(dst2d, msgs):
    call = pl.kernel(
        _scatter_body,
        out_type=jax.ShapeDtypeStruct((NC, ACC_N, D), jnp.float32),
        mesh=_mesh,
        scratch_types=[
            pltpu.VMEM((1, 128), jnp.int32),
            pltpu.VMEM((1, 128), jnp.int32),
            pltpu.VMEM((SCHUNK, D), jnp.float32),
            pltpu.VMEM((SCHUNK, D), jnp.float32),
            pltpu.SemaphoreType.DMA,
            pltpu.SemaphoreType.DMA,
            pltpu.VMEM_SHARED((ACC_N, D), jnp.float32),
        ],
    )
    return call(dst2d, msgs)


def kernel(tokens_ids, edge_index, W_embed, W_msg0, W_self0, W_edge0,
           W_msg1, W_self1, W_edge1):
    T = tokens_ids.shape[0]
    E = edge_index.shape[1]
    V = W_embed.shape[0]
    EP = ((E + NW * GCHUNK - 1) // (NW * GCHUNK)) * (NW * GCHUNK)

    t0 = tokens_ids[:, 0].astype(jnp.int32)
    t1 = tokens_ids[:, 1].astype(jnp.int32)
    pad_n = (jnp.arange(NP - N, dtype=jnp.int32) * 37) % V
    pad_e = (jnp.arange(EP - E, dtype=jnp.int32) * 37) % V
    n0 = jnp.concatenate([t0[:N], pad_n]).reshape(NP // 128, 128)
    n1 = jnp.concatenate([t1[:N], pad_n]).reshape(NP // 128, 128)
    e0 = jnp.concatenate([t0[N:], pad_e]).reshape(EP // 128, 128)
    e1 = jnp.concatenate([t1[N:], pad_e]).reshape(EP // 128, 128)

    src = edge_index[0].astype(jnp.int32)
    dst = edge_index[1].astype(jnp.int32)
    pad_src = jnp.arange(EP - E, dtype=jnp.int32) % N
    pad_dst = N + (jnp.arange(EP - E, dtype=jnp.int32) % (ACC_N - N))
    src2d = jnp.concatenate([src, pad_src]).reshape(EP // 128, 128)
    dst2d = jnp.concatenate([dst, pad_dst]).reshape(EP // 128, 128)

    nf_pad, ef = _sc_emb(n0, n1, e0, e1, W_embed, EP)

    g = _sc_gather(src2d, nf_pad, EP)
    msgs, ef = _edge_tc(g, ef, W_msg0, W_edge0, E)
    agg = _sc_scatter(dst2d, msgs)
    nf = _node_tc(nf_pad, W_self0, agg)

    g = _sc_gather(src2d, nf, EP)
    msgs, allfeats = _edge_tc_final(g, ef, W_msg1, W_edge1, E, T)
    agg = _sc_scatter(dst2d, msgs)
    return _node_tc_final(allfeats, nf, W_self1, agg)
